# baseline (speedup 1.0000x reference)
import functools

import jax
import jax.numpy as jnp
from jax import lax
from jax.experimental import pallas as pl
from jax.experimental.pallas import tpu as pltpu
from jax.experimental.pallas import tpu_sc as plsc

_B = 16384
_D = 32
_FF = 13
_NF = 26
_NR = _NF + 1
_NCOLS = _FF + _NF
_VOCAB = 100000

_NC = 2
_NS = 16
_NW = _NC * _NS
_BW = _B // _NW
_CB = 128
_NSUB = _BW // _CB
_CR = _CB * _NR
_IR = _CR // 128


def _precompute(features, W, b):
    BS = 2048

    def body(f_ref, w_ref, b_ref, d_ref, i_ref):
        x = f_ref[:, :_FF].astype(jnp.float32)
        d_ref[...] = (
            lax.dot_general(
                x, w_ref[...], (((1,), (1,)), ((), ())),
                preferred_element_type=jnp.float32,
            )
            + b_ref[...]
        )
        f26 = lax.broadcasted_iota(jnp.int32, (BS, _NF), 1)
        tok = f_ref[:, _FF:] + f26 * _VOCAB
        i_ref[...] = jnp.concatenate(
            [jnp.zeros((BS, 1), jnp.int32), tok], axis=1
        )

    return pl.pallas_call(
        body,
        grid=(_B // BS,),
        in_specs=[
            pl.BlockSpec((BS, _NCOLS), lambda i: (i, 0)),
            pl.BlockSpec((_D, _FF), lambda i: (0, 0)),
            pl.BlockSpec((1, _D), lambda i: (0, 0)),
        ],
        out_specs=[
            pl.BlockSpec((BS, _D), lambda i: (i, 0)),
            pl.BlockSpec((BS, _NR), lambda i: (i, 0)),
        ],
        out_shape=[
            jax.ShapeDtypeStruct((_B, _D), jnp.float32),
            jax.ShapeDtypeStruct((_B, _NR), jnp.int32),
        ],
    )(features, W, b.reshape(1, _D))


def _sc_assemble(idx_rows, dense, table, dloc):
    mesh = plsc.VectorSubcoreMesh(core_axis_name="c", subcore_axis_name="s")

    NP = 16
    PS = _BW * _NR // NP

    @functools.partial(
        pl.kernel,
        mesh=mesh,
        compiler_params=pltpu.CompilerParams(use_tc_tiling_on_sc=False),
        out_type=jax.ShapeDtypeStruct((_B * _NR, _D), jnp.float32),
        scratch_types=[
            pltpu.VMEM((PS,), jnp.int32),
            pltpu.VMEM((PS,), jnp.int32),
            pltpu.VMEM((PS, _D), jnp.float32),
            pltpu.VMEM((PS, _D), jnp.float32),
            pltpu.VMEM((_BW, _D), jnp.float32),
            pltpu.VMEM((_BW,), jnp.int32),
            pltpu.SemaphoreType.DMA,
            pltpu.SemaphoreType.DMA,
            pltpu.SemaphoreType.DMA,
            pltpu.SemaphoreType.DMA,
            pltpu.SemaphoreType.DMA,
        ],
    )
    def k(idx_hbm, dense_hbm, table_hbm, dloc_hbm, out_hbm,
          idx_v0, idx_v1, gbuf0, gbuf1, dvals, dloc_v,
          gsem0, gsem1, wsem0, wsem1, ssem):
        wid = lax.axis_index("s") * _NC + lax.axis_index("c")
        base = wid * _BW
        row0 = base * _NR
        idx_v = (idx_v0, idx_v1)
        gbuf = (gbuf0, gbuf1)
        gsem = (gsem0, gsem1)
        wsem = (wsem0, wsem1)

        def fire_gather(p):
            r = p & 1
            pltpu.sync_copy(idx_hbm.at[pl.ds(row0 + p * PS, PS)], idx_v[r])
            pltpu.async_copy(table_hbm.at[idx_v[r]], gbuf[r], gsem[r])

        def write_piece(p):
            r = p & 1
            pltpu.async_copy(
                gbuf[r], out_hbm.at[pl.ds(row0 + p * PS, PS)], wsem[r]
            )

        def wait_write(p):
            r = p & 1
            pltpu.make_async_copy(
                gbuf[r], out_hbm.at[pl.ds(row0 + p * PS, PS)], wsem[r]
            ).wait()

        fire_gather(0)
        for p in range(NP):
            r = p & 1
            if p + 1 < NP:
                if p >= 1:
                    wait_write(p - 1)
                fire_gather(p + 1)
            pltpu.make_async_copy(
                table_hbm.at[idx_v[r]], gbuf[r], gsem[r]
            ).wait()
            write_piece(p)
        wait_write(NP - 2)
        wait_write(NP - 1)

        pltpu.sync_copy(dense_hbm.at[pl.ds(base, _BW)], dvals)
        pltpu.sync_copy(dloc_hbm.at[wid], dloc_v)
        pltpu.async_copy(dvals, out_hbm.at[dloc_v], ssem).wait()

    return k(idx_rows, dense, table, dloc)


def kernel(original_features, table, W, b):
    dense, idx_ext = _precompute(original_features, W, b)
    idx_rows = idx_ext.reshape(_B * _NR)
    dloc = (jnp.arange(_B, dtype=jnp.int32) * _NR).reshape(_NW, _BW)
    out2d = _sc_assemble(idx_rows, dense, table, dloc)
    return out2d.reshape(_B, _NR, _D)


# speedup vs baseline: 1.0118x; 1.0118x over previous
import functools

import jax
import jax.numpy as jnp
from jax import lax
from jax.experimental import pallas as pl
from jax.experimental.pallas import tpu as pltpu
from jax.experimental.pallas import tpu_sc as plsc

_B = 16384
_D = 32
_FF = 13
_NF = 26
_NR = _NF + 1
_NCOLS = _FF + _NF
_VOCAB = 100000

_NC = 2
_NS = 16
_NW = _NC * _NS
_BW = _B // _NW
_CB = 128
_NSUB = _BW // _CB
_CR = _CB * _NR
_IR = _CR // 128


def _precompute(features, W, b):
    BS = 2048

    def body(f_ref, w_ref, b_ref, d_ref, i_ref):
        x = f_ref[:, :_FF].astype(jnp.float32)
        d_ref[...] = (
            lax.dot_general(
                x, w_ref[...], (((1,), (1,)), ((), ())),
                preferred_element_type=jnp.float32,
            )
            + b_ref[...]
        )
        f26 = lax.broadcasted_iota(jnp.int32, (BS, _NF), 1)
        tok = f_ref[:, _FF:] + f26 * _VOCAB
        i_ref[...] = jnp.concatenate(
            [jnp.zeros((BS, 1), jnp.int32), tok], axis=1
        )

    return pl.pallas_call(
        body,
        grid=(_B // BS,),
        in_specs=[
            pl.BlockSpec((BS, _NCOLS), lambda i: (i, 0)),
            pl.BlockSpec((_D, _FF), lambda i: (0, 0)),
            pl.BlockSpec((1, _D), lambda i: (0, 0)),
        ],
        out_specs=[
            pl.BlockSpec((BS, _D), lambda i: (i, 0)),
            pl.BlockSpec((BS, _NR), lambda i: (i, 0)),
        ],
        out_shape=[
            jax.ShapeDtypeStruct((_B, _D), jnp.float32),
            jax.ShapeDtypeStruct((_B, _NR), jnp.int32),
        ],
    )(features, W, b.reshape(1, _D))


def _sc_assemble(idx_rows, dense, table, dloc):
    mesh = plsc.VectorSubcoreMesh(core_axis_name="c", subcore_axis_name="s")

    @functools.partial(
        pl.kernel,
        mesh=mesh,
        compiler_params=pltpu.CompilerParams(use_tc_tiling_on_sc=False),
        out_type=jax.ShapeDtypeStruct((_B * _NR, _D), jnp.float32),
        scratch_types=[
            pltpu.VMEM((_CR,), jnp.int32),
            pltpu.VMEM((_CR, _D), jnp.float32),
            pltpu.VMEM((_CB, _D), jnp.float32),
            pltpu.VMEM((128,), jnp.int32),
            pltpu.SemaphoreType.DMA,
            pltpu.SemaphoreType.DMA,
        ],
    )
    def k(idx_hbm, dense_hbm, table_hbm, dloc_hbm, out_hbm,
          idx_v, gbuf, dvals, dloc_v, gsem, ssem):
        wid = lax.axis_index("s") * _NC + lax.axis_index("c")
        base = wid * _BW

        for sub in range(_NSUB):
            b0 = base + sub * _CB
            pltpu.sync_copy(idx_hbm.at[pl.ds(b0 * _NR, _CR)], idx_v)
            pltpu.sync_copy(dloc_hbm.at[b0 // _CB], dloc_v)

            pltpu.async_copy(table_hbm.at[idx_v], gbuf, gsem).wait()

            pltpu.sync_copy(dense_hbm.at[pl.ds(b0, _CB)], dvals)
            pltpu.sync_copy(gbuf, out_hbm.at[pl.ds(b0 * _NR, _CR)])
            pltpu.async_copy(dvals, out_hbm.at[dloc_v], ssem).wait()

    return k(idx_rows, dense, table, dloc)


def kernel(original_features, table, W, b):
    dense, idx_ext = _precompute(original_features, W, b)
    idx_rows = idx_ext.reshape(_B * _NR)
    dloc = (jnp.arange(_B, dtype=jnp.int32) * _NR).reshape(_B // _CB, _CB)
    out2d = _sc_assemble(idx_rows, dense, table, dloc)
    return out2d.reshape(_B, _NR, _D)
